# Initial kernel scaffold; baseline (speedup 1.0000x reference)
import jax
import jax.numpy as jnp
from jax.experimental import pallas as pl


def kernel(x, edge_index, W1, b1, W2, b2, W3, b3):
    raise NotImplementedError("write your pallas kernel here")


# speedup vs baseline: 19.8045x; 19.8045x over previous
import functools

import jax
import jax.numpy as jnp
from jax import lax
from jax.experimental import pallas as pl
from jax.experimental.pallas import tpu as pltpu
from jax.experimental.pallas import tpu_sc as plsc

N = 10000
E = 320000
D_IN = 128
D_HID = 64
D_OUT = 128

NC = 2
NS = 16
EB = 125
E_ROWS = E // EB
CHUNK = 16

@functools.lru_cache(maxsize=None)
def _mesh():
  return plsc.VectorSubcoreMesh(
      core_axis_name="c", subcore_axis_name="s", num_cores=NC, num_subcores=NS)


def _deg_body(dst_hbm, zeros_hbm, ones_hbm, out_hbm, deg_sh, ones_v, idx_v):
  c = lax.axis_index("c")
  s = lax.axis_index("s")
  @pl.when(s == 0)
  def _():
    pltpu.sync_copy(zeros_hbm, deg_sh)
  pltpu.sync_copy(ones_hbm, ones_v)
  plsc.subcore_barrier()

  base = c * (E_ROWS // NC) + s * (E_ROWS // NC // NS)

  def chunk(g, _):
    pltpu.sync_copy(dst_hbm.at[pl.ds(base + g * CHUNK, CHUNK)], idx_v)

    def blk(j, _):
      pltpu.sync_copy(ones_v, deg_sh.at[idx_v.at[j]], add=True)
      return 0
    return lax.fori_loop(0, CHUNK, blk, 0)

  lax.fori_loop(0, (E_ROWS // NC // NS) // CHUNK, chunk, 0)
  plsc.subcore_barrier()

  @pl.when(s == 0)
  def _():
    pltpu.sync_copy(deg_sh, out_hbm.at[pl.ds(c * N, N)])


@functools.lru_cache(maxsize=None)
def _deg_kernel():
  return pl.kernel(
      _deg_body,
      out_type=jax.ShapeDtypeStruct((NC * N, 16), jnp.float32),
      mesh=_mesh(),
      compiler_params=pltpu.CompilerParams(use_tc_tiling_on_sc=False),
      scratch_types=[
          pltpu.VMEM_SHARED((N, 16), jnp.float32),
          pltpu.VMEM((EB, 16), jnp.float32),
          pltpu.VMEM((CHUNK, EB), jnp.int32),
      ],
  )


def _agg_body(feat_split, p_hbm, zeros_hbm, src_hbm, dst_hbm, out_hbm,
              p_sh, agg_sh, src_v, dst_v, rows_v):
  c = lax.axis_index("c")
  s = lax.axis_index("s")
  p_off = c * N if feat_split else 0
  @pl.when(s == 0)
  def _():
    pltpu.sync_copy(p_hbm.at[pl.ds(p_off, N)], p_sh)
    pltpu.sync_copy(zeros_hbm, agg_sh)
  plsc.subcore_barrier()

  if feat_split:
    base = s * (E_ROWS // NS)
    n_chunks = (E_ROWS // NS) // CHUNK
  else:
    base = c * (E_ROWS // NC) + s * (E_ROWS // NC // NS)
    n_chunks = (E_ROWS // NC // NS) // CHUNK

  def chunk(g, _):
    r = base + g * CHUNK
    pltpu.sync_copy(src_hbm.at[pl.ds(r, CHUNK)], src_v)
    pltpu.sync_copy(dst_hbm.at[pl.ds(r, CHUNK)], dst_v)

    def blk(j, _):
      pltpu.sync_copy(p_sh.at[src_v.at[j]], rows_v)
      pltpu.sync_copy(rows_v, agg_sh.at[dst_v.at[j]], add=True)
      return 0
    return lax.fori_loop(0, CHUNK, blk, 0)

  lax.fori_loop(0, n_chunks, chunk, 0)
  plsc.subcore_barrier()

  @pl.when(s == 0)
  def _():
    pltpu.sync_copy(agg_sh, out_hbm.at[pl.ds(c * N, N)])


@functools.lru_cache(maxsize=None)
def _make_agg_kernel(feat_split):
  return pl.kernel(
      functools.partial(_agg_body, feat_split),
      out_type=jax.ShapeDtypeStruct((NC * N, D_HID), jnp.float32),
      mesh=_mesh(),
      compiler_params=pltpu.CompilerParams(use_tc_tiling_on_sc=False),
      scratch_types=[
          pltpu.VMEM_SHARED((N, D_HID), jnp.float32),
          pltpu.VMEM_SHARED((N, D_HID), jnp.float32),
          pltpu.VMEM((CHUNK, EB), jnp.int32),
          pltpu.VMEM((CHUNK, EB), jnp.int32),
          pltpu.VMEM((EB, D_HID), jnp.float32),
      ],
  )


_BN = 1000
_GRID = N // _BN


def _k1_body(dega_ref, degb_ref, x_ref, w_ref, p_ref, dis_ref):
  deg = dega_ref[:, :1] + degb_ref[:, :1] + 1.0
  dis = lax.rsqrt(deg)
  dis_ref[...] = dis
  h = jnp.dot(x_ref[...], w_ref[...], preferred_element_type=jnp.float32)
  p_ref[...] = h * dis


def _tc1(degp, x, W1):
  return pl.pallas_call(
      _k1_body,
      grid=(_GRID,),
      in_specs=[
          pl.BlockSpec((_BN, 16), lambda i: (i, 0)),
          pl.BlockSpec((_BN, 16), lambda i: (i + _GRID, 0)),
          pl.BlockSpec((_BN, D_IN), lambda i: (i, 0)),
          pl.BlockSpec((D_IN, D_HID), lambda i: (0, 0)),
      ],
      out_specs=[
          pl.BlockSpec((_BN, D_HID), lambda i: (i, 0)),
          pl.BlockSpec((_BN, 1), lambda i: (i, 0)),
      ],
      out_shape=[
          jax.ShapeDtypeStruct((N, D_HID), jnp.float32),
          jax.ShapeDtypeStruct((N, 1), jnp.float32),
      ],
  )(degp, degp, x, W1)


def _k2_body(agga_ref, aggb_ref, p_ref, dis_ref, b_ref, w_ref, out_ref):
  dis = dis_ref[...]
  z = dis * (agga_ref[...] + aggb_ref[...] + p_ref[...]) + b_ref[...]
  z = jnp.maximum(z, 0.0)
  h = jnp.dot(z, w_ref[...], preferred_element_type=jnp.float32)
  out_ref[...] = h * dis


def _tc2(agg, p, dis, b, W):
  return pl.pallas_call(
      _k2_body,
      grid=(_GRID,),
      in_specs=[
          pl.BlockSpec((_BN, D_HID), lambda i: (i, 0)),
          pl.BlockSpec((_BN, D_HID), lambda i: (i + _GRID, 0)),
          pl.BlockSpec((_BN, D_HID), lambda i: (i, 0)),
          pl.BlockSpec((_BN, 1), lambda i: (i, 0)),
          pl.BlockSpec((1, D_HID), lambda i: (0, 0)),
          pl.BlockSpec((D_HID, D_HID), lambda i: (0, 0)),
      ],
      out_specs=pl.BlockSpec((_BN, D_HID), lambda i: (i, 0)),
      out_shape=jax.ShapeDtypeStruct((N, D_HID), jnp.float32),
  )(agg, agg, p, dis, b, W)


def _k3_body(agga_ref, aggb_ref, p_ref, dis_ref, b_ref, w_ref, out_ref):
  dis = dis_ref[...]
  z = dis * (agga_ref[...] + aggb_ref[...] + p_ref[...]) + b_ref[...]
  z = jnp.maximum(z, 0.0)
  h = jnp.dot(z, w_ref[0], preferred_element_type=jnp.float32)
  out_ref[...] = h * dis


def _tc3(agg, p, dis, b, W3s):
  return pl.pallas_call(
      _k3_body,
      grid=(_GRID, NC),
      in_specs=[
          pl.BlockSpec((_BN, D_HID), lambda i, c: (i, 0)),
          pl.BlockSpec((_BN, D_HID), lambda i, c: (i + _GRID, 0)),
          pl.BlockSpec((_BN, D_HID), lambda i, c: (i, 0)),
          pl.BlockSpec((_BN, 1), lambda i, c: (i, 0)),
          pl.BlockSpec((1, D_HID), lambda i, c: (0, 0)),
          pl.BlockSpec((1, D_HID, D_HID), lambda i, c: (c, 0, 0)),
      ],
      out_specs=pl.BlockSpec((_BN, D_HID), lambda i, c: (c * _GRID + i, 0)),
      out_shape=jax.ShapeDtypeStruct((NC * N, D_HID), jnp.float32),
  )(agg, agg, p, dis, b, W3s)


def _k4_body(agga_ref, aggb_ref, pa_ref, pb_ref, dis_ref, b_ref, out_ref):
  lo = agga_ref[...] + pa_ref[...]
  hi = aggb_ref[...] + pb_ref[...]
  out_ref[...] = dis_ref[...] * jnp.concatenate([lo, hi], axis=1) + b_ref[...]


def _tc4(agg3, p3, dis, b3):
  return pl.pallas_call(
      _k4_body,
      grid=(_GRID,),
      in_specs=[
          pl.BlockSpec((_BN, D_HID), lambda i: (i, 0)),
          pl.BlockSpec((_BN, D_HID), lambda i: (i + _GRID, 0)),
          pl.BlockSpec((_BN, D_HID), lambda i: (i, 0)),
          pl.BlockSpec((_BN, D_HID), lambda i: (i + _GRID, 0)),
          pl.BlockSpec((_BN, 1), lambda i: (i, 0)),
          pl.BlockSpec((1, D_OUT), lambda i: (0, 0)),
      ],
      out_specs=pl.BlockSpec((_BN, D_OUT), lambda i: (i, 0)),
      out_shape=jax.ShapeDtypeStruct((N, D_OUT), jnp.float32),
  )(agg3, agg3, p3, p3, dis, b3)


def kernel(x, edge_index, W1, b1, W2, b2, W3, b3):
  ei = edge_index.astype(jnp.int32)
  src = ei[0].reshape(E_ROWS, EB)
  dst = ei[1].reshape(E_ROWS, EB)

  zeros64 = jnp.zeros((N, D_HID), jnp.float32)
  zeros16 = jnp.zeros((N, 16), jnp.float32)
  ones125 = jnp.ones((EB, 16), jnp.float32)

  degp = _deg_kernel()(dst, zeros16, ones125)

  p1, dis = _tc1(degp, x, W1)
  agg1 = _make_agg_kernel(False)(p1, zeros64, src, dst)
  p2 = _tc2(agg1, p1, dis, b1.reshape(1, D_HID), W2)
  agg2 = _make_agg_kernel(False)(p2, zeros64, src, dst)
  W3s = W3.reshape(D_HID, NC, D_HID).transpose(1, 0, 2)
  p3 = _tc3(agg2, p2, dis, b2.reshape(1, D_HID), W3s)
  agg3 = _make_agg_kernel(True)(p3, zeros64, src, dst)
  return _tc4(agg3, p3, dis, b3.reshape(1, D_OUT))


# speedup vs baseline: 25.4617x; 1.2856x over previous
import functools

import jax
import jax.numpy as jnp
from jax import lax
from jax.experimental import pallas as pl
from jax.experimental.pallas import tpu as pltpu
from jax.experimental.pallas import tpu_sc as plsc

N = 10000
E = 320000
D_IN = 128
D_HID = 64
D_OUT = 128

NC = 2
NS = 16
EB = 125
E_ROWS = E // EB
CHUNK = 16
CROWS = 80

@functools.lru_cache(maxsize=None)
def _mesh():
  return plsc.VectorSubcoreMesh(
      core_axis_name="c", subcore_axis_name="s", num_cores=NC, num_subcores=NS)


def _deg_body(dst_hbm, zeros_hbm, ones_hbm, out_hbm, deg_sh, ones_v, idx_v):
  c = lax.axis_index("c")
  s = lax.axis_index("s")
  @pl.when(s == 0)
  def _():
    pltpu.sync_copy(zeros_hbm, deg_sh)
  pltpu.sync_copy(ones_hbm, ones_v)
  plsc.subcore_barrier()

  base = c * (E_ROWS // NC) + s * (E_ROWS // NC // NS)

  def chunk(g, _):
    pltpu.sync_copy(dst_hbm.at[pl.ds(base + g * CHUNK, CHUNK)], idx_v)

    def blk(j, _):
      pltpu.sync_copy(ones_v, deg_sh.at[idx_v.at[j]], add=True)
      return 0
    return lax.fori_loop(0, CHUNK, blk, 0)

  lax.fori_loop(0, (E_ROWS // NC // NS) // CHUNK, chunk, 0)
  plsc.subcore_barrier()

  @pl.when(s == 0)
  def _():
    pltpu.sync_copy(deg_sh, out_hbm.at[pl.ds(c * N, N)])


@functools.lru_cache(maxsize=None)
def _deg_kernel():
  return pl.kernel(
      _deg_body,
      out_type=jax.ShapeDtypeStruct((NC * N, 16), jnp.float32),
      mesh=_mesh(),
      compiler_params=pltpu.CompilerParams(use_tc_tiling_on_sc=False),
      scratch_types=[
          pltpu.VMEM_SHARED((N, 16), jnp.float32),
          pltpu.VMEM((EB, 16), jnp.float32),
          pltpu.VMEM((CHUNK, EB), jnp.int32),
      ],
  )


def _agg_body(feat_split, p_hbm, zeros_hbm, src_hbm, dst_hbm, out_hbm,
              p_sh, agg_sh, src_v, dst_v, rows0, rows1, sem0, sem1):
  c = lax.axis_index("c")
  s = lax.axis_index("s")
  if feat_split:
    nb = E_ROWS // NS
    base = s * nb
  else:
    nb = E_ROWS // NC // NS
    base = c * (E_ROWS // NC) + s * nb

  p_off = c * N if feat_split else 0
  for t in range(nb // CROWS):
    pltpu.sync_copy(src_hbm.at[pl.ds(base + t * CROWS, CROWS)], src_v)
    pltpu.sync_copy(dst_hbm.at[pl.ds(base + t * CROWS, CROWS)], dst_v)
    if t == 0:
      @pl.when(s == 0)
      def _():
        pltpu.sync_copy(p_hbm.at[pl.ds(p_off, N)], p_sh)
        pltpu.sync_copy(zeros_hbm, agg_sh)
      plsc.subcore_barrier()

    pltpu.async_copy(p_sh.at[src_v.at[0]], rows0, sem0)

    def pair(k, _):
      j0 = 2 * k
      j1 = j0 + 1
      pltpu.async_copy(p_sh.at[src_v.at[j1]], rows1, sem1)
      pltpu.make_async_copy(p_sh.at[src_v.at[j0]], rows0, sem0).wait()
      pltpu.sync_copy(rows0, agg_sh.at[dst_v.at[j0]], add=True)

      @pl.when(k < CROWS // 2 - 1)
      def _():
        pltpu.async_copy(p_sh.at[src_v.at[j0 + 2]], rows0, sem0)

      pltpu.make_async_copy(p_sh.at[src_v.at[j1]], rows1, sem1).wait()
      pltpu.sync_copy(rows1, agg_sh.at[dst_v.at[j1]], add=True)
      return 0

    lax.fori_loop(0, CROWS // 2, pair, 0)
  plsc.subcore_barrier()

  @pl.when(s == 0)
  def _():
    pltpu.sync_copy(agg_sh, out_hbm.at[pl.ds(c * N, N)])


@functools.lru_cache(maxsize=None)
def _make_agg_kernel(feat_split):
  return pl.kernel(
      functools.partial(_agg_body, feat_split),
      out_type=jax.ShapeDtypeStruct((NC * N, D_HID), jnp.float32),
      mesh=_mesh(),
      compiler_params=pltpu.CompilerParams(use_tc_tiling_on_sc=False),
      scratch_types=[
          pltpu.VMEM_SHARED((N, D_HID), jnp.float32),
          pltpu.VMEM_SHARED((N, D_HID), jnp.float32),
          pltpu.VMEM((CROWS, EB), jnp.int32),
          pltpu.VMEM((CROWS, EB), jnp.int32),
          pltpu.VMEM((EB, D_HID), jnp.float32),
          pltpu.VMEM((EB, D_HID), jnp.float32),
          pltpu.SemaphoreType.DMA,
          pltpu.SemaphoreType.DMA,
      ],
  )


_BN = 1000
_GRID = N // _BN


def _k1_body(dega_ref, degb_ref, x_ref, w_ref, p_ref, dis_ref):
  deg = dega_ref[:, :1] + degb_ref[:, :1] + 1.0
  dis = lax.rsqrt(deg)
  dis_ref[...] = dis
  h = jnp.dot(x_ref[...], w_ref[...], preferred_element_type=jnp.float32)
  p_ref[...] = h * dis


def _tc1(degp, x, W1):
  return pl.pallas_call(
      _k1_body,
      grid=(_GRID,),
      in_specs=[
          pl.BlockSpec((_BN, 16), lambda i: (i, 0)),
          pl.BlockSpec((_BN, 16), lambda i: (i + _GRID, 0)),
          pl.BlockSpec((_BN, D_IN), lambda i: (i, 0)),
          pl.BlockSpec((D_IN, D_HID), lambda i: (0, 0)),
      ],
      out_specs=[
          pl.BlockSpec((_BN, D_HID), lambda i: (i, 0)),
          pl.BlockSpec((_BN, 1), lambda i: (i, 0)),
      ],
      out_shape=[
          jax.ShapeDtypeStruct((N, D_HID), jnp.float32),
          jax.ShapeDtypeStruct((N, 1), jnp.float32),
      ],
  )(degp, degp, x, W1)


def _k2_body(agga_ref, aggb_ref, p_ref, dis_ref, b_ref, w_ref, out_ref):
  dis = dis_ref[...]
  z = dis * (agga_ref[...] + aggb_ref[...] + p_ref[...]) + b_ref[...]
  z = jnp.maximum(z, 0.0)
  h = jnp.dot(z, w_ref[...], preferred_element_type=jnp.float32)
  out_ref[...] = h * dis


def _tc2(agg, p, dis, b, W):
  return pl.pallas_call(
      _k2_body,
      grid=(_GRID,),
      in_specs=[
          pl.BlockSpec((_BN, D_HID), lambda i: (i, 0)),
          pl.BlockSpec((_BN, D_HID), lambda i: (i + _GRID, 0)),
          pl.BlockSpec((_BN, D_HID), lambda i: (i, 0)),
          pl.BlockSpec((_BN, 1), lambda i: (i, 0)),
          pl.BlockSpec((1, D_HID), lambda i: (0, 0)),
          pl.BlockSpec((D_HID, D_HID), lambda i: (0, 0)),
      ],
      out_specs=pl.BlockSpec((_BN, D_HID), lambda i: (i, 0)),
      out_shape=jax.ShapeDtypeStruct((N, D_HID), jnp.float32),
  )(agg, agg, p, dis, b, W)


def _k3_body(agga_ref, aggb_ref, p_ref, dis_ref, b_ref, w_ref, out_ref):
  dis = dis_ref[...]
  z = dis * (agga_ref[...] + aggb_ref[...] + p_ref[...]) + b_ref[...]
  z = jnp.maximum(z, 0.0)
  h = jnp.dot(z, w_ref[0], preferred_element_type=jnp.float32)
  out_ref[...] = h * dis


def _tc3(agg, p, dis, b, W3s):
  return pl.pallas_call(
      _k3_body,
      grid=(_GRID, NC),
      in_specs=[
          pl.BlockSpec((_BN, D_HID), lambda i, c: (i, 0)),
          pl.BlockSpec((_BN, D_HID), lambda i, c: (i + _GRID, 0)),
          pl.BlockSpec((_BN, D_HID), lambda i, c: (i, 0)),
          pl.BlockSpec((_BN, 1), lambda i, c: (i, 0)),
          pl.BlockSpec((1, D_HID), lambda i, c: (0, 0)),
          pl.BlockSpec((1, D_HID, D_HID), lambda i, c: (c, 0, 0)),
      ],
      out_specs=pl.BlockSpec((_BN, D_HID), lambda i, c: (c * _GRID + i, 0)),
      out_shape=jax.ShapeDtypeStruct((NC * N, D_HID), jnp.float32),
  )(agg, agg, p, dis, b, W3s)


def _k4_body(agga_ref, aggb_ref, pa_ref, pb_ref, dis_ref, b_ref, out_ref):
  lo = agga_ref[...] + pa_ref[...]
  hi = aggb_ref[...] + pb_ref[...]
  out_ref[...] = dis_ref[...] * jnp.concatenate([lo, hi], axis=1) + b_ref[...]


def _tc4(agg3, p3, dis, b3):
  return pl.pallas_call(
      _k4_body,
      grid=(_GRID,),
      in_specs=[
          pl.BlockSpec((_BN, D_HID), lambda i: (i, 0)),
          pl.BlockSpec((_BN, D_HID), lambda i: (i + _GRID, 0)),
          pl.BlockSpec((_BN, D_HID), lambda i: (i, 0)),
          pl.BlockSpec((_BN, D_HID), lambda i: (i + _GRID, 0)),
          pl.BlockSpec((_BN, 1), lambda i: (i, 0)),
          pl.BlockSpec((1, D_OUT), lambda i: (0, 0)),
      ],
      out_specs=pl.BlockSpec((_BN, D_OUT), lambda i: (i, 0)),
      out_shape=jax.ShapeDtypeStruct((N, D_OUT), jnp.float32),
  )(agg3, agg3, p3, p3, dis, b3)


def kernel(x, edge_index, W1, b1, W2, b2, W3, b3):
  ei = edge_index.astype(jnp.int32)
  src = ei[0].reshape(E_ROWS, EB)
  dst = ei[1].reshape(E_ROWS, EB)

  zeros64 = jnp.zeros((N, D_HID), jnp.float32)
  zeros16 = jnp.zeros((N, 16), jnp.float32)
  ones125 = jnp.ones((EB, 16), jnp.float32)

  degp = _deg_kernel()(dst, zeros16, ones125)

  p1, dis = _tc1(degp, x, W1)
  agg1 = _make_agg_kernel(False)(p1, zeros64, src, dst)
  p2 = _tc2(agg1, p1, dis, b1.reshape(1, D_HID), W2)
  agg2 = _make_agg_kernel(False)(p2, zeros64, src, dst)
  W3s = W3.reshape(D_HID, NC, D_HID).transpose(1, 0, 2)
  p3 = _tc3(agg2, p2, dis, b2.reshape(1, D_HID), W3s)
  agg3 = _make_agg_kernel(True)(p3, zeros64, src, dst)
  return _tc4(agg3, p3, dis, b3.reshape(1, D_OUT))


# speedup vs baseline: 31.5080x; 1.2375x over previous
import functools

import jax
import jax.numpy as jnp
from jax import lax
from jax.experimental import pallas as pl
from jax.experimental.pallas import tpu as pltpu
from jax.experimental.pallas import tpu_sc as plsc

N = 10000
E = 320000
D_IN = 128
D_HID = 64
D_OUT = 128

NC = 2
NS = 16
EB = 125
E_ROWS = E // EB
NB = E_ROWS // NC // NS


@functools.lru_cache(maxsize=None)
def _mesh():
  return plsc.VectorSubcoreMesh(
      core_axis_name="c", subcore_axis_name="s", num_cores=NC, num_subcores=NS)


def _deg_body(dst_hbm, zeros_hbm, ones_hbm, out_hbm, deg_sh, ones_v, idx_v):
  c = lax.axis_index("c")
  s = lax.axis_index("s")
  base = c * (E_ROWS // NC) + s * NB
  pltpu.sync_copy(dst_hbm.at[pl.ds(base, NB)], idx_v)
  @pl.when(s == 0)
  def _():
    pltpu.sync_copy(zeros_hbm, deg_sh)
  pltpu.sync_copy(ones_hbm, ones_v)
  plsc.subcore_barrier()

  def blk(j, _):
    pltpu.sync_copy(ones_v, deg_sh.at[idx_v.at[j]], add=True)
    return 0

  lax.fori_loop(0, NB, blk, 0)
  plsc.subcore_barrier()

  @pl.when(s == 0)
  def _():
    pltpu.sync_copy(deg_sh, out_hbm.at[pl.ds(c * N, N)])


@functools.lru_cache(maxsize=None)
def _deg_kernel():
  return pl.kernel(
      _deg_body,
      out_type=jax.ShapeDtypeStruct((NC * N, 16), jnp.float32),
      mesh=_mesh(),
      compiler_params=pltpu.CompilerParams(use_tc_tiling_on_sc=False),
      scratch_types=[
          pltpu.VMEM_SHARED((N, 16), jnp.float32),
          pltpu.VMEM((EB, 16), jnp.float32),
          pltpu.VMEM((NB, EB), jnp.int32),
      ],
  )


def _agg_body(d, crows, p_hbm, zeros_hbm, src_hbm, dst_hbm, out_hbm,
              agg_sh, src_v, dst_v, rows0, rows1, sem0, sem1):
  c = lax.axis_index("c")
  s = lax.axis_index("s")
  base = c * (E_ROWS // NC) + s * NB

  for t in range(NB // crows):
    pltpu.sync_copy(src_hbm.at[pl.ds(base + t * crows, crows)], src_v)
    pltpu.sync_copy(dst_hbm.at[pl.ds(base + t * crows, crows)], dst_v)
    if t == 0:
      @pl.when(s == 0)
      def _():
        pltpu.sync_copy(zeros_hbm, agg_sh)
      plsc.subcore_barrier()

    pltpu.async_copy(p_hbm.at[src_v.at[0]], rows0, sem0)

    def pair(k, _):
      j0 = 2 * k
      j1 = j0 + 1
      pltpu.async_copy(p_hbm.at[src_v.at[j1]], rows1, sem1)
      pltpu.make_async_copy(p_hbm.at[src_v.at[j0]], rows0, sem0).wait()
      pltpu.sync_copy(rows0, agg_sh.at[dst_v.at[j0]], add=True)

      @pl.when(k < crows // 2 - 1)
      def _():
        pltpu.async_copy(p_hbm.at[src_v.at[j0 + 2]], rows0, sem0)

      pltpu.make_async_copy(p_hbm.at[src_v.at[j1]], rows1, sem1).wait()
      pltpu.sync_copy(rows1, agg_sh.at[dst_v.at[j1]], add=True)
      return 0

    lax.fori_loop(0, crows // 2, pair, 0)
  plsc.subcore_barrier()

  @pl.when(s == 0)
  def _():
    pltpu.sync_copy(agg_sh, out_hbm.at[pl.ds(c * N, N)])


@functools.lru_cache(maxsize=None)
def _make_agg_kernel(d, crows):
  return pl.kernel(
      functools.partial(_agg_body, d, crows),
      out_type=jax.ShapeDtypeStruct((NC * N, d), jnp.float32),
      mesh=_mesh(),
      compiler_params=pltpu.CompilerParams(use_tc_tiling_on_sc=False),
      scratch_types=[
          pltpu.VMEM_SHARED((N, d), jnp.float32),
          pltpu.VMEM((crows, EB), jnp.int32),
          pltpu.VMEM((crows, EB), jnp.int32),
          pltpu.VMEM((EB, d), jnp.float32),
          pltpu.VMEM((EB, d), jnp.float32),
          pltpu.SemaphoreType.DMA,
          pltpu.SemaphoreType.DMA,
      ],
  )


_BN = 1000
_GRID = N // _BN


def _k1_body(dega_ref, degb_ref, x_ref, w_ref, p_ref, dis_ref):
  deg = dega_ref[:, :1] + degb_ref[:, :1] + 1.0
  dis = lax.rsqrt(deg)
  dis_ref[...] = dis
  h = jnp.dot(x_ref[...], w_ref[...], preferred_element_type=jnp.float32)
  p_ref[...] = h * dis


def _tc1(degp, x, W1):
  return pl.pallas_call(
      _k1_body,
      grid=(_GRID,),
      in_specs=[
          pl.BlockSpec((_BN, 16), lambda i: (i, 0)),
          pl.BlockSpec((_BN, 16), lambda i: (i + _GRID, 0)),
          pl.BlockSpec((_BN, D_IN), lambda i: (i, 0)),
          pl.BlockSpec((D_IN, D_HID), lambda i: (0, 0)),
      ],
      out_specs=[
          pl.BlockSpec((_BN, D_HID), lambda i: (i, 0)),
          pl.BlockSpec((_BN, 1), lambda i: (i, 0)),
      ],
      out_shape=[
          jax.ShapeDtypeStruct((N, D_HID), jnp.float32),
          jax.ShapeDtypeStruct((N, 1), jnp.float32),
      ],
  )(degp, degp, x, W1)


def _k2_body(agga_ref, aggb_ref, p_ref, dis_ref, b_ref, w_ref, out_ref):
  dis = dis_ref[...]
  z = dis * (agga_ref[...] + aggb_ref[...] + p_ref[...]) + b_ref[...]
  z = jnp.maximum(z, 0.0)
  h = jnp.dot(z, w_ref[...], preferred_element_type=jnp.float32)
  out_ref[...] = h * dis


def _tc2(agg, p, dis, b, W, d_out):
  return pl.pallas_call(
      _k2_body,
      grid=(_GRID,),
      in_specs=[
          pl.BlockSpec((_BN, D_HID), lambda i: (i, 0)),
          pl.BlockSpec((_BN, D_HID), lambda i: (i + _GRID, 0)),
          pl.BlockSpec((_BN, D_HID), lambda i: (i, 0)),
          pl.BlockSpec((_BN, 1), lambda i: (i, 0)),
          pl.BlockSpec((1, D_HID), lambda i: (0, 0)),
          pl.BlockSpec((D_HID, d_out), lambda i: (0, 0)),
      ],
      out_specs=pl.BlockSpec((_BN, d_out), lambda i: (i, 0)),
      out_shape=jax.ShapeDtypeStruct((N, d_out), jnp.float32),
  )(agg, agg, p, dis, b, W)


def _k4_body(agga_ref, aggb_ref, p_ref, dis_ref, b_ref, out_ref):
  out_ref[...] = dis_ref[...] * (
      agga_ref[...] + aggb_ref[...] + p_ref[...]) + b_ref[...]


def _tc4(agg3, p3, dis, b3):
  return pl.pallas_call(
      _k4_body,
      grid=(_GRID,),
      in_specs=[
          pl.BlockSpec((_BN, D_OUT), lambda i: (i, 0)),
          pl.BlockSpec((_BN, D_OUT), lambda i: (i + _GRID, 0)),
          pl.BlockSpec((_BN, D_OUT), lambda i: (i, 0)),
          pl.BlockSpec((_BN, 1), lambda i: (i, 0)),
          pl.BlockSpec((1, D_OUT), lambda i: (0, 0)),
      ],
      out_specs=pl.BlockSpec((_BN, D_OUT), lambda i: (i, 0)),
      out_shape=jax.ShapeDtypeStruct((N, D_OUT), jnp.float32),
  )(agg3, agg3, p3, dis, b3)


def kernel(x, edge_index, W1, b1, W2, b2, W3, b3):
  ei = edge_index.astype(jnp.int32)
  src = ei[0].reshape(E_ROWS, EB)
  dst = ei[1].reshape(E_ROWS, EB)

  zeros64 = jnp.zeros((N, D_HID), jnp.float32)
  zeros128 = jnp.zeros((N, D_OUT), jnp.float32)
  zeros16 = jnp.zeros((N, 16), jnp.float32)
  ones125 = jnp.ones((EB, 16), jnp.float32)

  degp = _deg_kernel()(dst, zeros16, ones125)

  p1, dis = _tc1(degp, x, W1)
  agg1 = _make_agg_kernel(D_HID, 80)(p1, zeros64, src, dst)
  p2 = _tc2(agg1, p1, dis, b1.reshape(1, D_HID), W2, D_HID)
  agg2 = _make_agg_kernel(D_HID, 80)(p2, zeros64, src, dst)
  p3 = _tc2(agg2, p2, dis, b2.reshape(1, D_HID), W3, D_OUT)
  agg3 = _make_agg_kernel(D_OUT, 40)(p3, zeros128, src, dst)
  return _tc4(agg3, p3, dis, b3.reshape(1, D_OUT))
